# baseline (speedup 1.0000x reference)
import jax
import jax.numpy as jnp
from jax import lax
from jax.experimental import pallas as pl
from jax.experimental.pallas import tpu as pltpu
from jax.experimental.pallas import tpu_sc as plsc

D = 64
SCALE = 8.0
NC = 2
NS = 16
NW = NC * NS
B_TOT = 4096 * 200
B_PER_W = B_TOT // NW
CHUNK = 512
NBUF = 2
NCHUNK = B_PER_W // CHUNK
GSPLIT = CHUNK // 128
LANES = 16


def _emb_body(table_hbm, idx_hbm, out_hbm,
              idx0, idx1, rows0, rows1, gsem0, gsem1, ssem0, ssem1):
    idx_v = (idx0, idx1)
    rows_v = (rows0, rows1)
    gsems = (gsem0, gsem1)
    ssems = (ssem0, ssem1)

    wid = lax.axis_index("s") * NC + lax.axis_index("c")
    base = wid * B_PER_W

    def gather_descs(b):
        return [
            pltpu.make_async_copy(
                table_hbm.at[idx_v[b].at[pl.ds(j * 128, 128)]],
                rows_v[b].at[pl.ds(j * 128, 128), :],
                gsems[b],
            )
            for j in range(GSPLIT)
        ]

    def fetch(g, b):
        off = base + g * CHUNK
        pltpu.sync_copy(idx_hbm.at[pl.ds(off, CHUNK)], idx_v[b])
        for d_ in gather_descs(b):
            d_.start()

    def drain_gather(b):
        for d_ in gather_descs(b):
            d_.wait()

    def store_desc(g, b):
        off = base + g * CHUNK
        return pltpu.make_async_copy(
            rows_v[b], out_hbm.at[pl.ds(off, CHUNK), :], ssems[b])

    def scale(b):
        r = rows_v[b]

        @pl.loop(0, CHUNK, unroll=8)
        def _(i):
            for j in range(D // LANES):
                sl = (i, pl.ds(j * LANES, LANES))
                r[sl] = r[sl] * SCALE

    for b in range(NBUF):
        fetch(b, b)

    @pl.loop(0, NCHUNK - NBUF, step=NBUF)
    def _(g0):
        for b in range(NBUF):
            drain_gather(b)
            scale(b)
            store_desc(g0 + b, b).start()
        for b in range(NBUF):
            store_desc(g0 + b, b).wait()
            fetch(g0 + b + NBUF, b)

    for b in range(NBUF):
        g = NCHUNK - NBUF + b
        drain_gather(b)
        scale(b)
        store_desc(g, b).start()
    for b in range(NBUF):
        store_desc(NCHUNK - NBUF + b, b).wait()


@jax.jit
def _emb_lookup(table, idx):
    mesh = plsc.VectorSubcoreMesh(core_axis_name="c", subcore_axis_name="s")
    f = pl.kernel(
        _emb_body,
        out_type=jax.ShapeDtypeStruct((B_TOT, D), jnp.float32),
        mesh=mesh,
        scratch_types=[
            pltpu.VMEM((CHUNK,), jnp.int32),
            pltpu.VMEM((CHUNK,), jnp.int32),
            pltpu.VMEM((CHUNK, D), jnp.float32),
            pltpu.VMEM((CHUNK, D), jnp.float32),
            pltpu.SemaphoreType.DMA,
            pltpu.SemaphoreType.DMA,
            pltpu.SemaphoreType.DMA,
            pltpu.SemaphoreType.DMA,
        ],
        compiler_params=pltpu.CompilerParams(use_tc_tiling_on_sc=False),
    )
    return f(table, idx)


def kernel(x, table):
    idx = x.reshape(-1)
    out = _emb_lookup(table, idx)
    return out.reshape(x.shape + (D,))
